# baseline (speedup 1.0000x reference)
import functools

import jax
import jax.numpy as jnp
from jax import lax
from jax.experimental import pallas as pl
from jax.experimental.pallas import tpu as pltpu
from jax.experimental.pallas import tpu_sc as plsc

B, S, D = 4, 4096, 1024
NW = 32
S_PER_W = S // NW
CHUNK = 16
N_CHUNKS = S_PER_W // CHUNK
LANES = 16
DV = D // LANES

_GATHER_DNUMS = lax.GatherDimensionNumbers(
    offset_dims=(), collapsed_slice_dims=(0,), start_index_map=(0,))


def _bcast_lane(vec, j):
    idx = jnp.full((LANES, 1), j, dtype=jnp.int32)
    return lax.gather(vec, idx, _GATHER_DNUMS, slice_sizes=(1,),
                      mode=lax.GatherScatterMode.PROMISE_IN_BOUNDS)


def _sc_body(x_hbm, pe_hbm, rt_hbm, ot_hbm, ri_hbm, oi_hbm, out_hbm,
             rt_v, ot_v, pe_v, out_v, ri_v, oi_v):
    wid = lax.axis_index("s") * 2 + lax.axis_index("c")
    s_base = wid * S_PER_W

    pltpu.sync_copy(rt_hbm, rt_v)
    pltpu.sync_copy(ot_hbm, ot_v)
    for bb in range(B):
        pltpu.sync_copy(ri_hbm.at[bb, pl.ds(s_base, S_PER_W)], ri_v.at[bb])
        pltpu.sync_copy(oi_hbm.at[bb, pl.ds(s_base, S_PER_W)], oi_v.at[bb])

    iota = lax.broadcasted_iota(jnp.int32, (LANES,), 0)

    def step(i, carry):
        cc = i // B
        b = i % B
        s0 = s_base + cc * CHUNK

        @pl.when(b == 0)
        def _():
            pltpu.sync_copy(pe_hbm.at[pl.ds(s0, CHUNK)], pe_v)

        pltpu.sync_copy(x_hbm.at[b, pl.ds(s0, CHUNK)], out_v)

        rvec = ri_v[b, pl.ds(cc * CHUNK, CHUNK)]
        ovec = oi_v[b, pl.ds(cc * CHUNK, CHUNK)]

        def token(j, carry2):
            r_spl = _bcast_lane(rvec, j)
            o_spl = _bcast_lane(ovec, j)
            for d in range(DV):
                col = iota + (d * LANES)
                rv = plsc.load_gather(rt_v, [r_spl, col])
                ov = plsc.load_gather(ot_v, [o_spl, col])
                pv = pe_v[j, pl.ds(d * LANES, LANES)]
                plsc.addupdate(out_v.at[j, pl.ds(d * LANES, LANES)],
                               pv + rv + ov)
            return carry2

        lax.fori_loop(0, CHUNK, token, 0, unroll=False)
        pltpu.sync_copy(out_v, out_hbm.at[b, pl.ds(s0, CHUNK)])
        return carry

    lax.fori_loop(0, N_CHUNKS * B, step, 0, unroll=False)


@jax.jit
def _sc_call(x, pe, rt, ot, ri, oi):
    mesh = plsc.VectorSubcoreMesh(core_axis_name="c", subcore_axis_name="s")
    kern = functools.partial(
        pl.kernel,
        mesh=mesh,
        out_type=jax.ShapeDtypeStruct((B, S, D), jnp.float32),
        compiler_params=pltpu.CompilerParams(
            use_tc_tiling_on_sc=False, needs_layout_passes=False),
        scratch_types=[
            pltpu.VMEM((16, D), jnp.float32),
            pltpu.VMEM((32, D), jnp.float32),
            pltpu.VMEM((CHUNK, D), jnp.float32),
            pltpu.VMEM((CHUNK, D), jnp.float32),
            pltpu.VMEM((B, S_PER_W), jnp.int32),
            pltpu.VMEM((B, S_PER_W), jnp.int32),
        ],
    )(_sc_body)
    return kern(x, pe, rt, ot, ri, oi)


def kernel(x, pe, round_table, op_table, round_indices, operation_indices):
    ri = round_indices.astype(jnp.int32)
    oi = operation_indices.astype(jnp.int32)
    pe_used = pe[:S]
    return _sc_call(x, pe_used, round_table, op_table, ri, oi)


# speedup vs baseline: 1.6986x; 1.6986x over previous
import functools

import jax
import jax.numpy as jnp
from jax import lax
from jax.experimental import pallas as pl
from jax.experimental.pallas import tpu as pltpu
from jax.experimental.pallas import tpu_sc as plsc

B, S, D = 4, 4096, 1024
NW = 32
S_PER_W = S // NW
CHUNK = 16
N_CHUNKS = S_PER_W // CHUNK
N_STEPS = N_CHUNKS * B
LANES = 16
DV = D // LANES


def _comb_body(rt_ref, ot_ref, out_ref):
    rt = rt_ref[...]
    ot = ot_ref[...]
    out_ref[...] = (rt[:, None, :] + ot[None, :, :]).reshape(16 * 32, D)


@jax.jit
def _make_comb(rt, ot):
    return pl.pallas_call(
        _comb_body,
        out_shape=jax.ShapeDtypeStruct((16 * 32, D), jnp.float32),
    )(rt, ot)


def _sc_body(x_hbm, pe_hbm, comb_hbm, ri_hbm, oi_hbm, out_hbm,
             pe_v, buf_v, rows_v, ri_v, oi_v,
             sem_x0, sem_x1, sem_g0, sem_g1, sem_o0, sem_o1,
             sem_p0, sem_p1):
    wid = lax.axis_index("s") * 2 + lax.axis_index("c")
    s_base = wid * S_PER_W
    sem_x = (sem_x0, sem_x1)
    sem_g = (sem_g0, sem_g1)
    sem_o = (sem_o0, sem_o1)
    sem_p = (sem_p0, sem_p1)

    for bb in range(B):
        pltpu.sync_copy(ri_hbm.at[bb, pl.ds(s_base, S_PER_W)], ri_v.at[bb])
        pltpu.sync_copy(oi_hbm.at[bb, pl.ds(s_base, S_PER_W)], oi_v.at[bb])

    def civ(b, c):
        rvec = ri_v[b, pl.ds(c * CHUNK, CHUNK)]
        ovec = oi_v[b, pl.ds(c * CHUNK, CHUNK)]
        return rvec * 32 + ovec

    def x_copy(b, c, slot):
        s0 = s_base + c * CHUNK
        return pltpu.make_async_copy(
            x_hbm.at[b, pl.ds(s0, CHUNK)], buf_v.at[slot], sem_x[slot])

    def g_copy(b, c, slot):
        return pltpu.make_async_copy(
            comb_hbm.at[civ(b, c)], rows_v.at[slot], sem_g[slot])

    def o_copy(b, c, slot):
        s0 = s_base + c * CHUNK
        return pltpu.make_async_copy(
            buf_v.at[slot], out_hbm.at[b, pl.ds(s0, CHUNK)], sem_o[slot])

    def p_copy(c, slot):
        s0 = s_base + c * CHUNK
        return pltpu.make_async_copy(
            pe_hbm.at[pl.ds(s0, CHUNK)], pe_v.at[slot], sem_p[slot])

    x_copy(0, 0, 0).start()
    g_copy(0, 0, 0).start()
    p_copy(0, 0).start()

    def super_step(ii, carry):
        for k in range(8):
            b = k % 4
            c = 2 * ii + k // 4
            cur = k % 2
            nxt = (k + 1) % 2
            pslot = k // 4

            if k == 0:
                @pl.when(ii > 0)
                def _():
                    o_copy(3, 2 * ii - 1, nxt).wait()
                x_copy(1, 2 * ii, nxt).start()
                g_copy(1, 2 * ii, nxt).start()
                p_copy(2 * ii + 1, 1).start()
                p_copy(2 * ii, 0).wait()
            elif k == 7:
                o_copy(2, 2 * ii + 1, nxt).wait()
                @pl.when(ii < N_STEPS // 8 - 1)
                def _():
                    x_copy(0, 2 * ii + 2, nxt).start()
                    g_copy(0, 2 * ii + 2, nxt).start()
            else:
                bp = (k - 1) % 4
                cp = 2 * ii + (k - 1) // 4
                bn = (k + 1) % 4
                cn = 2 * ii + (k + 1) // 4
                o_copy(bp, cp, nxt).wait()
                x_copy(bn, cn, nxt).start()
                g_copy(bn, cn, nxt).start()
                if k == 4:
                    @pl.when(ii < N_STEPS // 8 - 1)
                    def _():
                        p_copy(2 * ii + 2, 0).start()
                    p_copy(2 * ii + 1, 1).wait()

            x_copy(b, c, cur).wait()
            g_copy(b, c, cur).wait()

            @plsc.parallel_loop(0, CHUNK)
            def _(j):
                for d in range(DV):
                    dd = pl.ds(d * LANES, LANES)
                    plsc.addupdate(buf_v.at[cur, j, dd],
                                   pe_v[pslot, j, dd] + rows_v[cur, j, dd])

            o_copy(b, c, cur).start()
        return carry

    lax.fori_loop(0, N_STEPS // 8, super_step, 0, unroll=False)
    o_copy(3, N_CHUNKS - 1, 1).wait()


@jax.jit
def _sc_call(x, pe, comb, ri, oi):
    mesh = plsc.VectorSubcoreMesh(core_axis_name="c", subcore_axis_name="s")
    kern = functools.partial(
        pl.kernel,
        mesh=mesh,
        out_type=jax.ShapeDtypeStruct((B, S, D), jnp.float32),
        compiler_params=pltpu.CompilerParams(
            use_tc_tiling_on_sc=False, needs_layout_passes=False),
        scratch_types=[
            pltpu.VMEM((2, CHUNK, D), jnp.float32),
            pltpu.VMEM((2, CHUNK, D), jnp.float32),
            pltpu.VMEM((2, CHUNK, D), jnp.float32),
            pltpu.VMEM((B, S_PER_W), jnp.int32),
            pltpu.VMEM((B, S_PER_W), jnp.int32),
        ] + [pltpu.SemaphoreType.DMA] * 8,
    )(_sc_body)
    return kern(x, pe, comb, ri, oi)


def kernel(x, pe, round_table, op_table, round_indices, operation_indices):
    ri = round_indices.astype(jnp.int32)
    oi = operation_indices.astype(jnp.int32)
    comb = _make_comb(round_table, op_table)
    return _sc_call(x, pe[:S], comb, ri, oi)


# speedup vs baseline: 3.5146x; 2.0691x over previous
import functools

import jax
import jax.numpy as jnp
from jax import lax
from jax.experimental import pallas as pl
from jax.experimental.pallas import tpu as pltpu
from jax.experimental.pallas import tpu_sc as plsc

B, S, D = 4, 4096, 1024
NW = 32
S_PER_W = S // NW
CHUNK = 16
N_CHUNKS = S_PER_W // CHUNK
N_STEPS = N_CHUNKS * B
LANES = 16
NT = D // 128
VPT = 128 // LANES


def _comb_body(rt_ref, ot_ref, out_ref):
    for a in range(NT):
        lanes = pl.ds(a * 128, 128)
        out_ref[:, :, a, :] = (rt_ref[:, lanes][:, None, :]
                               + ot_ref[:, lanes][None, :, :])


@jax.jit
def _make_comb(rt, ot):
    out4 = pl.pallas_call(
        _comb_body,
        out_shape=jax.ShapeDtypeStruct((16, 32, NT, 128), jnp.float32),
    )(rt, ot)
    return out4.reshape(512, NT, 128)


def _sc_body(x_hbm, pe_hbm, comb_hbm, civ_hbm, out_hbm,
             pe_v, buf_v, rows_v, civ_v,
             sem_x0, sem_x1, sem_g0, sem_g1, sem_o0, sem_o1,
             sem_p0, sem_p1):
    wid = lax.axis_index("s") * 2 + lax.axis_index("c")
    sem_x = (sem_x0, sem_x1)
    sem_g = (sem_g0, sem_g1)
    sem_o = (sem_o0, sem_o1)
    sem_p = (sem_p0, sem_p1)

    for bb in range(B):
        pltpu.sync_copy(civ_hbm.at[bb, wid], civ_v.at[bb])

    def srow(c):
        return wid * (S_PER_W // 8) + c * (CHUNK // 8)

    def x_copy(b, c, slot):
        return pltpu.make_async_copy(
            x_hbm.at[b, pl.ds(srow(c), CHUNK // 8)], buf_v.at[slot],
            sem_x[slot])

    def g_copy(b, c, slot):
        gidx = civ_v[b, pl.ds(c * CHUNK, CHUNK)]
        return pltpu.make_async_copy(
            comb_hbm.at[gidx], rows_v.at[slot], sem_g[slot])

    def o_copy(b, c, slot):
        return pltpu.make_async_copy(
            buf_v.at[slot], out_hbm.at[b, pl.ds(srow(c), CHUNK // 8)],
            sem_o[slot])

    def p_copy(c, slot):
        return pltpu.make_async_copy(
            pe_hbm.at[pl.ds(srow(c), CHUNK // 8)], pe_v.at[slot],
            sem_p[slot])

    x_copy(0, 0, 0).start()
    g_copy(0, 0, 0).start()
    p_copy(0, 0).start()

    def super_step(ii, carry):
        for k in range(8):
            b = k % 4
            c = 2 * ii + k // 4
            cur = k % 2
            nxt = (k + 1) % 2
            pslot = k // 4

            if k == 0:
                @pl.when(ii > 0)
                def _():
                    o_copy(3, 2 * ii - 1, nxt).wait()
                x_copy(1, 2 * ii, nxt).start()
                g_copy(1, 2 * ii, nxt).start()
                p_copy(2 * ii + 1, 1).start()
                p_copy(2 * ii, 0).wait()
            elif k == 7:
                o_copy(2, 2 * ii + 1, nxt).wait()
                @pl.when(ii < N_STEPS // 8 - 1)
                def _():
                    x_copy(0, 2 * ii + 2, nxt).start()
                    g_copy(0, 2 * ii + 2, nxt).start()
            else:
                bp = (k - 1) % 4
                cp = 2 * ii + (k - 1) // 4
                bn = (k + 1) % 4
                cn = 2 * ii + (k + 1) // 4
                o_copy(bp, cp, nxt).wait()
                x_copy(bn, cn, nxt).start()
                g_copy(bn, cn, nxt).start()
                if k == 4:
                    @pl.when(ii < N_STEPS // 8 - 1)
                    def _():
                        p_copy(2 * ii + 2, 0).start()
                    p_copy(2 * ii + 1, 1).wait()

            x_copy(b, c, cur).wait()
            g_copy(b, c, cur).wait()

            @plsc.parallel_loop(0, CHUNK)
            def _(j):
                jr = j // 8
                js = j % 8
                for a in range(NT):
                    for v in range(VPT):
                        dd = pl.ds(v * LANES, LANES)
                        plsc.addupdate(
                            buf_v.at[cur, jr, a, js, dd],
                            pe_v[pslot, jr, a, js, dd]
                            + rows_v[cur, j, a, dd])

            o_copy(b, c, cur).start()
        return carry

    lax.fori_loop(0, N_STEPS // 8, super_step, 0, unroll=False)
    o_copy(3, N_CHUNKS - 1, 1).wait()


@jax.jit
def _sc_call(x5, pe5, comb, civ):
    mesh = plsc.VectorSubcoreMesh(core_axis_name="c", subcore_axis_name="s")
    kern = functools.partial(
        pl.kernel,
        mesh=mesh,
        out_type=jax.ShapeDtypeStruct((B, S // 8, NT, 8, 128), jnp.float32),
        compiler_params=pltpu.CompilerParams(
            use_tc_tiling_on_sc=False, needs_layout_passes=False),
        scratch_types=[
            pltpu.VMEM((2, CHUNK // 8, NT, 8, 128), jnp.float32),
            pltpu.VMEM((2, CHUNK // 8, NT, 8, 128), jnp.float32),
            pltpu.VMEM((2, CHUNK, NT, 128), jnp.float32),
            pltpu.VMEM((B, 128), jnp.int32),
        ] + [pltpu.SemaphoreType.DMA] * 8,
    )(_sc_body)
    return kern(x5, pe5, comb, civ)


def kernel(x, pe, round_table, op_table, round_indices, operation_indices):
    ri = round_indices.astype(jnp.int32)
    oi = operation_indices.astype(jnp.int32)
    civ = (ri * 32 + oi).reshape(B, NW, 128)
    comb = _make_comb(round_table, op_table)
    x5 = x.reshape(B, S // 8, 8, NT, 128).transpose(0, 1, 3, 2, 4)
    pe5 = pe.reshape(pe.shape[0] // 8, 8, NT, 128).transpose(0, 2, 1, 3)
    out5 = _sc_call(x5, pe5, comb, civ)
    return out5.transpose(0, 1, 3, 2, 4).reshape(B, S, D)
